# baseline (speedup 1.0000x reference)
import functools

import jax
import jax.numpy as jnp
from jax import lax
from jax.experimental import pallas as pl
from jax.experimental.pallas import tpu as pltpu
from jax.experimental.pallas import tpu_sc as plsc

_N = 10000
_IN = 128
_HID = 16
_E = 320000

_CHUNK = 1024
_EPT = 20480
_NMACRO = _EPT // _CHUNK
_EROWS = 16 * _NMACRO
_NPAD = 10240
_ZR = _NPAD // 16
_PR = _N // 16

_BLK = 2000

_sc_mesh = plsc.VectorSubcoreMesh(core_axis_name="c", subcore_axis_name="s")

_f32 = jnp.float32


@functools.partial(
    pl.kernel,
    out_type=(jax.ShapeDtypeStruct((2, _N, _HID), _f32),
              jax.ShapeDtypeStruct((2, _NPAD, _HID), _f32)),
    mesh=_sc_mesh,
    scratch_types=[
        pltpu.VMEM((_NMACRO, _CHUNK), jnp.int32),
        pltpu.VMEM((_NMACRO, _CHUNK), jnp.int32),
        pltpu.VMEM((_CHUNK, _HID), _f32),
        pltpu.VMEM((_ZR, _HID), _f32),
        pltpu.VMEM((_PR, _HID), _f32),
        pltpu.VMEM((_PR, _HID), _f32),
        pltpu.VMEM_SHARED((_NPAD, _HID), _f32),
        pltpu.VMEM_SHARED((_NPAD, _HID), _f32),
        pltpu.SemaphoreType.DMA,
        pltpu.SemaphoreType.DMA,
    ],
    compiler_params=pltpu.CompilerParams(use_tc_tiling_on_sc=False),
)
def _gnn_sc(xr_hbm, xs_hbm, src_hbm, dst_hbm, h_hbm, a2_hbm,
            sidx, didx, rows, zbuf, hbuf, xsbuf, acc1, acc2, gsem, ssem):
    cid = lax.axis_index("c")
    sid = lax.axis_index("s")
    xr = xr_hbm.at[cid]
    xs = xs_hbm.at[cid]
    src = src_hbm.at[cid]
    dst = dst_hbm.at[cid]
    hout = h_hbm.at[cid]
    a2out = a2_hbm.at[cid]

    zero = jnp.zeros((_HID,), _f32)

    @pl.loop(0, _ZR)
    def _(i):
        zbuf[i, :] = zero

    pltpu.sync_copy(zbuf, acc1.at[pl.ds(sid * _ZR, _ZR)])
    pltpu.sync_copy(zbuf, acc2.at[pl.ds(sid * _ZR, _ZR)])
    plsc.subcore_barrier()

    def seg(x_src_ref, acc):
        base = sid * _NMACRO
        pltpu.sync_copy(src.at[pl.ds(base, _NMACRO)], sidx)
        pltpu.sync_copy(dst.at[pl.ds(base, _NMACRO)], didx)

        @pl.loop(0, _NMACRO)
        def _(m):
            pltpu.async_copy(x_src_ref.at[sidx.at[m]], rows, gsem).wait()
            pltpu.sync_copy(rows, acc.at[didx.at[m]], add=True)

    seg(xr, acc1)
    plsc.subcore_barrier()

    p0 = sid * _PR
    pltpu.sync_copy(acc1.at[pl.ds(p0, _PR)], hbuf)
    pltpu.sync_copy(xs.at[pl.ds(p0, _PR)], xsbuf)

    @pl.loop(0, _PR)
    def _(i):
        hbuf[i, :] = jnp.maximum(hbuf[i, :] + xsbuf[i, :], 0.0)

    pltpu.sync_copy(hbuf, hout.at[pl.ds(p0, _PR)])
    plsc.subcore_barrier()

    seg(hout, acc2)
    plsc.subcore_barrier()
    pltpu.sync_copy(acc2.at[pl.ds(sid * _ZR, _ZR)],
                    a2out.at[pl.ds(sid * _ZR, _ZR)])


def _mlp1_body(xp_ref, xl_ref, wrp, wsp, wrl, wsl, bp, bl,
               xrp_ref, xsp_ref, xrl_ref, xsl_ref):
    xp = xp_ref[...]
    xl = xl_ref[...]
    xrp_ref[...] = jnp.dot(xp, wrp[...], preferred_element_type=_f32)
    xsp_ref[...] = jnp.dot(xp, wsp[...], preferred_element_type=_f32) + bp[...]
    xrl_ref[...] = jnp.dot(xl, wrl[...], preferred_element_type=_f32)
    xsl_ref[...] = jnp.dot(xl, wsl[...], preferred_element_type=_f32) + bl[...]


def _head_body(a2p, hp, a2l, hl, wrp2, wsp2, wrl2, wsl2, b2p, b2l,
               winp, winl, bin_, wout, bout, out_ref):
    ap_t = jnp.dot(wrp2[...], winp[...], preferred_element_type=_f32)
    ap_b = jnp.dot(wsp2[...], winp[...], preferred_element_type=_f32)
    al_t = jnp.dot(wrl2[...], winl[...], preferred_element_type=_f32)
    al_b = jnp.dot(wsl2[...], winl[...], preferred_element_type=_f32)
    c = (jnp.dot(b2p[...], winp[...], preferred_element_type=_f32)
         + jnp.dot(b2l[...], winl[...], preferred_element_type=_f32)
         + bin_[...])
    a = (jnp.dot(a2p[...], ap_t, preferred_element_type=_f32)
         + jnp.dot(hp[...], ap_b, preferred_element_type=_f32)
         + jnp.dot(a2l[...], al_t, preferred_element_type=_f32)
         + jnp.dot(hl[...], al_b, preferred_element_type=_f32)
         + c)
    a = jnp.maximum(a, 0.0)
    out_ref[...] = jnp.tanh(
        jnp.dot(a, wout[...], preferred_element_type=_f32) + bout[...])


def _full(shape):
    return pl.BlockSpec(shape, lambda i: (0, 0))


def _rows(w):
    return pl.BlockSpec((_BLK, w), lambda i: (i, 0))


def _pad_edges(ei):
    ei = ei.astype(jnp.int32)
    npad = _EROWS * _CHUNK - _E
    src = jnp.concatenate([ei[0], jnp.zeros((npad,), jnp.int32)]).reshape(_EROWS, _CHUNK)
    dst = jnp.concatenate([ei[1], jnp.full((npad,), _N, jnp.int32)]).reshape(_EROWS, _CHUNK)
    return src, dst


def kernel(protein_data, protein_edge_index, ligand_data, ligand_edge_index,
           p_Wr1, p_Ws1, p_b1, p_Wr2, p_Ws2, p_b2,
           l_Wr1, l_Ws1, l_b1, l_Wr2, l_Ws2, l_b2,
           W_in, b_in, W_out, b_out):
    sp, dp = _pad_edges(protein_edge_index)
    sl, dl = _pad_edges(ligand_edge_index)
    src_st = jnp.stack([sp, sl])
    dst_st = jnp.stack([dp, dl])

    nblk = _N // _BLK
    o16 = jax.ShapeDtypeStruct((_N, _HID), _f32)

    xrp, xsp, xrl, xsl = pl.pallas_call(
        _mlp1_body,
        grid=(nblk,),
        in_specs=[_rows(_IN), _rows(_IN),
                  _full((_IN, _HID)), _full((_IN, _HID)),
                  _full((_IN, _HID)), _full((_IN, _HID)),
                  _full((1, _HID)), _full((1, _HID))],
        out_specs=[_rows(_HID)] * 4,
        out_shape=[o16] * 4,
    )(protein_data, ligand_data, p_Wr1, p_Ws1, l_Wr1, l_Ws1,
      p_b1.reshape(1, _HID), l_b1.reshape(1, _HID))

    xr_st = jnp.stack([xrp, xrl])
    xs_st = jnp.stack([xsp, xsl])

    h_st, a2_st = _gnn_sc(xr_st, xs_st, src_st, dst_st)
    hp, hl = h_st[0], h_st[1]
    a2p, a2l = a2_st[0, :_N], a2_st[1, :_N]

    ogcn = W_in.shape[0] // 2
    ahid = W_in.shape[1]
    act = W_out.shape[1]
    out = pl.pallas_call(
        _head_body,
        grid=(nblk,),
        in_specs=[_rows(_HID)] * 4 + [
            _full((_HID, ogcn)), _full((_HID, ogcn)),
            _full((_HID, ogcn)), _full((_HID, ogcn)),
            _full((1, ogcn)), _full((1, ogcn)),
            _full((ogcn, ahid)), _full((ogcn, ahid)),
            _full((1, ahid)), _full((ahid, act)), _full((1, act))],
        out_specs=_rows(act),
        out_shape=jax.ShapeDtypeStruct((_N, act), _f32),
    )(a2p, hp, a2l, hl, p_Wr2, p_Ws2, l_Wr2, l_Ws2,
      p_b2.reshape(1, ogcn), l_b2.reshape(1, ogcn),
      W_in[:ogcn], W_in[ogcn:], b_in.reshape(1, ahid), W_out,
      b_out.reshape(1, act))
    return out


# speedup vs baseline: 1.1005x; 1.1005x over previous
import functools

import jax
import jax.numpy as jnp
from jax import lax
from jax.experimental import pallas as pl
from jax.experimental.pallas import tpu as pltpu
from jax.experimental.pallas import tpu_sc as plsc

_N = 10000
_IN = 128
_HID = 16
_E = 320000

_CHUNK = 1024
_EPT = 20480
_NMACRO = _EPT // _CHUNK
_EROWS = 16 * _NMACRO
_NPAD = 10240
_ZR = _NPAD // 16
_PR = _N // 16

_BLK = 2000

_sc_mesh = plsc.VectorSubcoreMesh(core_axis_name="c", subcore_axis_name="s")

_f32 = jnp.float32


@functools.partial(
    pl.kernel,
    out_type=(jax.ShapeDtypeStruct((2, _N, _HID), _f32),
              jax.ShapeDtypeStruct((2, _NPAD, _HID), _f32)),
    mesh=_sc_mesh,
    scratch_types=[
        pltpu.VMEM((_NMACRO, _CHUNK), jnp.int32),
        pltpu.VMEM((_NMACRO, _CHUNK), jnp.int32),
        pltpu.VMEM((_CHUNK, _HID), _f32),
        pltpu.VMEM((_CHUNK, _HID), _f32),
        pltpu.VMEM((_ZR, _HID), _f32),
        pltpu.VMEM((_PR, _HID), _f32),
        pltpu.VMEM((_PR, _HID), _f32),
        pltpu.VMEM_SHARED((_NPAD, _HID), _f32),
        pltpu.VMEM_SHARED((_NPAD, _HID), _f32),
        pltpu.SemaphoreType.DMA,
        pltpu.SemaphoreType.DMA,
    ],
    compiler_params=pltpu.CompilerParams(use_tc_tiling_on_sc=False),
)
def _gnn_sc(xr_hbm, xs_hbm, src_hbm, dst_hbm, h_hbm, a2_hbm,
            sidx, didx, rows_a, rows_b, zbuf, hbuf, xsbuf, acc1, acc2,
            gsem, ssem):
    cid = lax.axis_index("c")
    sid = lax.axis_index("s")
    xr = xr_hbm.at[cid]
    xs = xs_hbm.at[cid]
    src = src_hbm.at[cid]
    dst = dst_hbm.at[cid]
    hout = h_hbm.at[cid]
    a2out = a2_hbm.at[cid]

    zero = jnp.zeros((_HID,), _f32)

    @pl.loop(0, _ZR)
    def _(i):
        zbuf[i, :] = zero

    pltpu.sync_copy(zbuf, acc1.at[pl.ds(sid * _ZR, _ZR)])
    pltpu.sync_copy(zbuf, acc2.at[pl.ds(sid * _ZR, _ZR)])
    plsc.subcore_barrier()

    def seg(x_src_ref, acc):
        base = sid * _NMACRO
        pltpu.sync_copy(src.at[pl.ds(base, _NMACRO)], sidx)
        pltpu.sync_copy(dst.at[pl.ds(base, _NMACRO)], didx)

        def g(m, buf):
            return pltpu.async_copy(x_src_ref.at[sidx.at[m]], buf, gsem)

        def s(m, buf):
            return pltpu.async_copy(buf, acc.at[didx.at[m]], ssem, add=True)

        @pl.loop(0, _NMACRO // 4)
        def _(mm):
            m0 = 4 * mm
            g_a = g(m0, rows_a)
            g_b = g(m0 + 1, rows_b)
            g_a.wait()
            s_a = s(m0, rows_a)
            s_a.wait()
            g_a2 = g(m0 + 2, rows_a)
            g_b.wait()
            s_b = s(m0 + 1, rows_b)
            s_b.wait()
            g_b2 = g(m0 + 3, rows_b)
            g_a2.wait()
            s_a2 = s(m0 + 2, rows_a)
            s_a2.wait()
            g_b2.wait()
            s_b2 = s(m0 + 3, rows_b)
            s_b2.wait()

    seg(xr, acc1)
    plsc.subcore_barrier()

    p0 = sid * _PR
    pltpu.sync_copy(acc1.at[pl.ds(p0, _PR)], hbuf)
    pltpu.sync_copy(xs.at[pl.ds(p0, _PR)], xsbuf)

    @pl.loop(0, _PR)
    def _(i):
        hbuf[i, :] = jnp.maximum(hbuf[i, :] + xsbuf[i, :], 0.0)

    pltpu.sync_copy(hbuf, hout.at[pl.ds(p0, _PR)])
    plsc.subcore_barrier()

    seg(hout, acc2)
    plsc.subcore_barrier()
    pltpu.sync_copy(acc2.at[pl.ds(sid * _ZR, _ZR)],
                    a2out.at[pl.ds(sid * _ZR, _ZR)])


def _mlp1_body(xp_ref, xl_ref, wrp, wsp, wrl, wsl, bp, bl,
               xrp_ref, xsp_ref, xrl_ref, xsl_ref):
    xp = xp_ref[...]
    xl = xl_ref[...]
    xrp_ref[...] = jnp.dot(xp, wrp[...], preferred_element_type=_f32)
    xsp_ref[...] = jnp.dot(xp, wsp[...], preferred_element_type=_f32) + bp[...]
    xrl_ref[...] = jnp.dot(xl, wrl[...], preferred_element_type=_f32)
    xsl_ref[...] = jnp.dot(xl, wsl[...], preferred_element_type=_f32) + bl[...]


def _head_body(a2p, hp, a2l, hl, wrp2, wsp2, wrl2, wsl2, b2p, b2l,
               winp, winl, bin_, wout, bout, out_ref):
    ap_t = jnp.dot(wrp2[...], winp[...], preferred_element_type=_f32)
    ap_b = jnp.dot(wsp2[...], winp[...], preferred_element_type=_f32)
    al_t = jnp.dot(wrl2[...], winl[...], preferred_element_type=_f32)
    al_b = jnp.dot(wsl2[...], winl[...], preferred_element_type=_f32)
    c = (jnp.dot(b2p[...], winp[...], preferred_element_type=_f32)
         + jnp.dot(b2l[...], winl[...], preferred_element_type=_f32)
         + bin_[...])
    a = (jnp.dot(a2p[...], ap_t, preferred_element_type=_f32)
         + jnp.dot(hp[...], ap_b, preferred_element_type=_f32)
         + jnp.dot(a2l[...], al_t, preferred_element_type=_f32)
         + jnp.dot(hl[...], al_b, preferred_element_type=_f32)
         + c)
    a = jnp.maximum(a, 0.0)
    out_ref[...] = jnp.tanh(
        jnp.dot(a, wout[...], preferred_element_type=_f32) + bout[...])


def _full(shape):
    return pl.BlockSpec(shape, lambda i: (0, 0))


def _rows(w):
    return pl.BlockSpec((_BLK, w), lambda i: (i, 0))


def _pad_edges(ei):
    ei = ei.astype(jnp.int32)
    npad = _EROWS * _CHUNK - _E
    src = jnp.concatenate([ei[0], jnp.zeros((npad,), jnp.int32)]).reshape(_EROWS, _CHUNK)
    dst = jnp.concatenate([ei[1], jnp.full((npad,), _N, jnp.int32)]).reshape(_EROWS, _CHUNK)
    return src, dst


def kernel(protein_data, protein_edge_index, ligand_data, ligand_edge_index,
           p_Wr1, p_Ws1, p_b1, p_Wr2, p_Ws2, p_b2,
           l_Wr1, l_Ws1, l_b1, l_Wr2, l_Ws2, l_b2,
           W_in, b_in, W_out, b_out):
    sp, dp = _pad_edges(protein_edge_index)
    sl, dl = _pad_edges(ligand_edge_index)
    src_st = jnp.stack([sp, sl])
    dst_st = jnp.stack([dp, dl])

    nblk = _N // _BLK
    o16 = jax.ShapeDtypeStruct((_N, _HID), _f32)

    xrp, xsp, xrl, xsl = pl.pallas_call(
        _mlp1_body,
        grid=(nblk,),
        in_specs=[_rows(_IN), _rows(_IN),
                  _full((_IN, _HID)), _full((_IN, _HID)),
                  _full((_IN, _HID)), _full((_IN, _HID)),
                  _full((1, _HID)), _full((1, _HID))],
        out_specs=[_rows(_HID)] * 4,
        out_shape=[o16] * 4,
    )(protein_data, ligand_data, p_Wr1, p_Ws1, l_Wr1, l_Ws1,
      p_b1.reshape(1, _HID), l_b1.reshape(1, _HID))

    xr_st = jnp.stack([xrp, xrl])
    xs_st = jnp.stack([xsp, xsl])

    h_st, a2_st = _gnn_sc(xr_st, xs_st, src_st, dst_st)
    hp, hl = h_st[0], h_st[1]
    a2p, a2l = a2_st[0, :_N], a2_st[1, :_N]

    ogcn = W_in.shape[0] // 2
    ahid = W_in.shape[1]
    act = W_out.shape[1]
    out = pl.pallas_call(
        _head_body,
        grid=(nblk,),
        in_specs=[_rows(_HID)] * 4 + [
            _full((_HID, ogcn)), _full((_HID, ogcn)),
            _full((_HID, ogcn)), _full((_HID, ogcn)),
            _full((1, ogcn)), _full((1, ogcn)),
            _full((ogcn, ahid)), _full((ogcn, ahid)),
            _full((1, ahid)), _full((ahid, act)), _full((1, act))],
        out_specs=_rows(act),
        out_shape=jax.ShapeDtypeStruct((_N, act), _f32),
    )(a2p, hp, a2l, hl, p_Wr2, p_Ws2, l_Wr2, l_Ws2,
      p_b2.reshape(1, ogcn), l_b2.reshape(1, ogcn),
      W_in[:ogcn], W_in[ogcn:], b_in.reshape(1, ahid), W_out,
      b_out.reshape(1, act))
    return out


# speedup vs baseline: 2.0128x; 1.8290x over previous
import functools

import jax
import jax.numpy as jnp
from jax import lax
from jax.experimental import pallas as pl
from jax.experimental.pallas import tpu as pltpu
from jax.experimental.pallas import tpu_sc as plsc

_N = 10000
_IN = 128
_HID = 16
_E = 320000

_CHUNK = 1000
_NMACRO = 20
_EROWS = _E // _CHUNK
_NPAD = 10240
_ZR = _NPAD // 16
_PR = _N // 16

_BLK = 2000

_sc_mesh = plsc.VectorSubcoreMesh(core_axis_name="c", subcore_axis_name="s")

_f32 = jnp.float32


@functools.partial(
    pl.kernel,
    out_type=(jax.ShapeDtypeStruct((_N, _HID), _f32),
              jax.ShapeDtypeStruct((_N, _HID), _f32),
              jax.ShapeDtypeStruct((_NPAD, _HID), _f32),
              jax.ShapeDtypeStruct((_NPAD, _HID), _f32)),
    mesh=_sc_mesh,
    scratch_types=[
        pltpu.VMEM((_NMACRO, _CHUNK), jnp.int32),
        pltpu.VMEM((_NMACRO, _CHUNK), jnp.int32),
        pltpu.VMEM((_CHUNK, _HID), _f32),
        pltpu.VMEM((_CHUNK, _HID), _f32),
        pltpu.VMEM((_ZR, _HID), _f32),
        pltpu.VMEM((_PR, _HID), _f32),
        pltpu.VMEM_SHARED((_NPAD, _HID), _f32),
        pltpu.VMEM_SHARED((_NPAD, _HID), _f32),
        pltpu.SemaphoreType.DMA,
        pltpu.SemaphoreType.DMA,
    ],
    compiler_params=pltpu.CompilerParams(use_tc_tiling_on_sc=False),
)
def _gnn_sc(xrp_hbm, xrl_hbm, xsp_hbm, xsl_hbm, ep_hbm, el_hbm,
            hp_hbm, hl_hbm, a2p_hbm, a2l_hbm,
            sidx, didx, rows_a, rows_b, hbuf, xsbuf, acc1, acc2, gsem, ssem):
    cid = lax.axis_index("c")
    sid = lax.axis_index("s")

    zero = jnp.zeros((_HID,), _f32)

    @pl.loop(0, _ZR)
    def _(i):
        hbuf[i, :] = zero

    pltpu.sync_copy(hbuf, acc1.at[pl.ds(sid * _ZR, _ZR)])
    pltpu.sync_copy(hbuf, acc2.at[pl.ds(sid * _ZR, _ZR)])
    plsc.subcore_barrier()

    def seg(x_src_ref, e_ref, acc):
        base = sid * _NMACRO
        pltpu.sync_copy(e_ref.at[0].at[pl.ds(base, _NMACRO)], sidx)
        pltpu.sync_copy(e_ref.at[1].at[pl.ds(base, _NMACRO)], didx)

        def g(m, buf):
            return pltpu.async_copy(x_src_ref.at[sidx.at[m]], buf, gsem)

        def s(m, buf):
            return pltpu.async_copy(buf, acc.at[didx.at[m]], ssem, add=True)

        @pl.loop(0, _NMACRO // 4)
        def _(mm):
            m0 = 4 * mm
            g_a = g(m0, rows_a)
            g_b = g(m0 + 1, rows_b)
            g_a.wait()
            s_a = s(m0, rows_a)
            s_a.wait()
            g_a2 = g(m0 + 2, rows_a)
            g_b.wait()
            s_b = s(m0 + 1, rows_b)
            s_b.wait()
            g_b2 = g(m0 + 3, rows_b)
            g_a2.wait()
            s_a2 = s(m0 + 2, rows_a)
            s_a2.wait()
            g_b2.wait()
            s_b2 = s(m0 + 3, rows_b)
            s_b2.wait()

    def branch(xr, xs, edges, hout, a2out):
        seg(xr, edges, acc1)
        plsc.subcore_barrier()

        p0 = sid * _PR
        pltpu.sync_copy(acc1.at[pl.ds(p0, _PR)], hbuf.at[pl.ds(0, _PR)])
        pltpu.sync_copy(xs.at[pl.ds(p0, _PR)], xsbuf)

        @pl.loop(0, _PR)
        def _(i):
            hbuf[i, :] = jnp.maximum(hbuf[i, :] + xsbuf[i, :], 0.0)

        pltpu.sync_copy(hbuf.at[pl.ds(0, _PR)], hout.at[pl.ds(p0, _PR)])
        plsc.subcore_barrier()

        seg(hout, edges, acc2)
        plsc.subcore_barrier()
        pltpu.sync_copy(acc2.at[pl.ds(sid * _ZR, _ZR)],
                        a2out.at[pl.ds(sid * _ZR, _ZR)])

    @pl.when(cid == 0)
    def _():
        branch(xrp_hbm, xsp_hbm, ep_hbm, hp_hbm, a2p_hbm)

    @pl.when(cid == 1)
    def _():
        branch(xrl_hbm, xsl_hbm, el_hbm, hl_hbm, a2l_hbm)


def _mlp1_body(xp_ref, xl_ref, wrp, wsp, wrl, wsl, bp, bl,
               xrp_ref, xsp_ref, xrl_ref, xsl_ref):
    xp = xp_ref[...]
    xl = xl_ref[...]
    xrp_ref[...] = jnp.dot(xp, wrp[...], preferred_element_type=_f32)
    xsp_ref[...] = jnp.dot(xp, wsp[...], preferred_element_type=_f32) + bp[...]
    xrl_ref[...] = jnp.dot(xl, wrl[...], preferred_element_type=_f32)
    xsl_ref[...] = jnp.dot(xl, wsl[...], preferred_element_type=_f32) + bl[...]


def _head_body(a2p, hp, a2l, hl, wrp2, wsp2, wrl2, wsl2, b2p, b2l,
               winp, winl, bin_, wout, bout, out_ref):
    ap_t = jnp.dot(wrp2[...], winp[...], preferred_element_type=_f32)
    ap_b = jnp.dot(wsp2[...], winp[...], preferred_element_type=_f32)
    al_t = jnp.dot(wrl2[...], winl[...], preferred_element_type=_f32)
    al_b = jnp.dot(wsl2[...], winl[...], preferred_element_type=_f32)
    c = (jnp.dot(b2p[...], winp[...], preferred_element_type=_f32)
         + jnp.dot(b2l[...], winl[...], preferred_element_type=_f32)
         + bin_[...])
    a = (jnp.dot(a2p[...], ap_t, preferred_element_type=_f32)
         + jnp.dot(hp[...], ap_b, preferred_element_type=_f32)
         + jnp.dot(a2l[...], al_t, preferred_element_type=_f32)
         + jnp.dot(hl[...], al_b, preferred_element_type=_f32)
         + c)
    a = jnp.maximum(a, 0.0)
    out_ref[...] = jnp.tanh(
        jnp.dot(a, wout[...], preferred_element_type=_f32) + bout[...])


def _full(shape):
    return pl.BlockSpec(shape, lambda i: (0, 0))


def _rows(w):
    return pl.BlockSpec((_BLK, w), lambda i: (i, 0))


def kernel(protein_data, protein_edge_index, ligand_data, ligand_edge_index,
           p_Wr1, p_Ws1, p_b1, p_Wr2, p_Ws2, p_b2,
           l_Wr1, l_Ws1, l_b1, l_Wr2, l_Ws2, l_b2,
           W_in, b_in, W_out, b_out):
    ep = protein_edge_index.astype(jnp.int32).reshape(2, _EROWS, _CHUNK)
    el = ligand_edge_index.astype(jnp.int32).reshape(2, _EROWS, _CHUNK)

    nblk = _N // _BLK
    o16 = jax.ShapeDtypeStruct((_N, _HID), _f32)

    xrp, xsp, xrl, xsl = pl.pallas_call(
        _mlp1_body,
        grid=(nblk,),
        in_specs=[_rows(_IN), _rows(_IN),
                  _full((_IN, _HID)), _full((_IN, _HID)),
                  _full((_IN, _HID)), _full((_IN, _HID)),
                  _full((1, _HID)), _full((1, _HID))],
        out_specs=[_rows(_HID)] * 4,
        out_shape=[o16] * 4,
    )(protein_data, ligand_data, p_Wr1, p_Ws1, l_Wr1, l_Ws1,
      p_b1.reshape(1, _HID), l_b1.reshape(1, _HID))

    hp, hl, a2p, a2l = _gnn_sc(xrp, xrl, xsp, xsl, ep, el)

    ogcn = W_in.shape[0] // 2
    ahid = W_in.shape[1]
    act = W_out.shape[1]
    out = pl.pallas_call(
        _head_body,
        grid=(nblk,),
        in_specs=[_rows(_HID)] * 4 + [
            _full((_HID, ogcn)), _full((_HID, ogcn)),
            _full((_HID, ogcn)), _full((_HID, ogcn)),
            _full((1, ogcn)), _full((1, ogcn)),
            _full((ogcn, ahid)), _full((ogcn, ahid)),
            _full((1, ahid)), _full((ahid, act)), _full((1, act))],
        out_specs=_rows(act),
        out_shape=jax.ShapeDtypeStruct((_N, act), _f32),
    )(a2p, hp, a2l, hl, p_Wr2, p_Ws2, l_Wr2, l_Ws2,
      p_b2.reshape(1, ogcn), l_b2.reshape(1, ogcn),
      W_in[:ogcn], W_in[ogcn:], b_in.reshape(1, ahid), W_out,
      b_out.reshape(1, act))
    return out


# speedup vs baseline: 2.0428x; 1.0149x over previous
import functools

import jax
import jax.numpy as jnp
from jax import lax
from jax.experimental import pallas as pl
from jax.experimental.pallas import tpu as pltpu
from jax.experimental.pallas import tpu_sc as plsc

_N = 10000
_IN = 128
_HID = 16
_E = 320000

_CHUNK = 1000
_NMACRO = 20
_EROWS = _E // _CHUNK
_NPAD = 10240
_ZR = _NPAD // 16
_PR = _N // 16

_BLK = 2000

_sc_mesh = plsc.VectorSubcoreMesh(core_axis_name="c", subcore_axis_name="s")

_f32 = jnp.float32


@functools.partial(
    pl.kernel,
    out_type=(jax.ShapeDtypeStruct((_N, _HID), _f32),
              jax.ShapeDtypeStruct((_N, _HID), _f32),
              jax.ShapeDtypeStruct((_NPAD, _HID), _f32),
              jax.ShapeDtypeStruct((_NPAD, _HID), _f32)),
    mesh=_sc_mesh,
    scratch_types=[
        pltpu.VMEM((_NMACRO, _CHUNK), jnp.int32),
        pltpu.VMEM((_NMACRO, _CHUNK), jnp.int32),
        pltpu.VMEM((_CHUNK, _HID), _f32),
        pltpu.VMEM((_CHUNK, _HID), _f32),
        pltpu.VMEM_SHARED((_NPAD, _HID), _f32),
        pltpu.VMEM_SHARED((_NPAD, _HID), _f32),
        pltpu.SemaphoreType.DMA,
        pltpu.SemaphoreType.DMA,
    ],
    compiler_params=pltpu.CompilerParams(use_tc_tiling_on_sc=False),
)
def _gnn_sc(xrp_hbm, xrl_hbm, xsp_hbm, xsl_hbm, ep_hbm, el_hbm,
            hp_hbm, hl_hbm, a2p_hbm, a2l_hbm,
            sidx, didx, rows_a, rows_b, acc1, acc2, gsem, ssem):
    cid = lax.axis_index("c")
    sid = lax.axis_index("s")

    hbuf = rows_a
    xsbuf = rows_b

    zero = jnp.zeros((_HID,), _f32)

    @pl.loop(0, _ZR)
    def _(i):
        hbuf[i, :] = zero

    pltpu.sync_copy(hbuf.at[pl.ds(0, _ZR)], acc1.at[pl.ds(sid * _ZR, _ZR)])
    pltpu.sync_copy(hbuf.at[pl.ds(0, _ZR)], acc2.at[pl.ds(sid * _ZR, _ZR)])
    plsc.subcore_barrier()

    def seg(x_src_ref, acc):
        def g(m, buf):
            return pltpu.async_copy(x_src_ref.at[sidx.at[m]], buf, gsem)

        def s(m, buf):
            return pltpu.async_copy(buf, acc.at[didx.at[m]], ssem, add=True)

        K = 5

        @pl.loop(0, _NMACRO // K)
        def _(mm):
            m0 = K * mm
            bufs = [rows_a, rows_b]
            inflight = {0: g(m0, bufs[0]), 1: g(m0 + 1, bufs[1])}
            for j in range(K):
                buf = bufs[j % 2]
                inflight[j].wait()
                s_h = s(m0 + j, buf)
                s_h.wait()
                if j + 2 < K:
                    inflight[j + 2] = g(m0 + j + 2, buf)

    def branch(xr, xs, edges, hout, a2out):
        base = sid * _NMACRO
        pltpu.sync_copy(edges.at[0].at[pl.ds(base, _NMACRO)], sidx)
        pltpu.sync_copy(edges.at[1].at[pl.ds(base, _NMACRO)], didx)

        seg(xr, acc1)
        plsc.subcore_barrier()

        p0 = sid * _PR
        pltpu.sync_copy(acc1.at[pl.ds(p0, _PR)], hbuf.at[pl.ds(0, _PR)])
        pltpu.sync_copy(xs.at[pl.ds(p0, _PR)], xsbuf.at[pl.ds(0, _PR)])

        @pl.loop(0, _PR)
        def _(i):
            hbuf[i, :] = jnp.maximum(hbuf[i, :] + xsbuf[i, :], 0.0)

        pltpu.sync_copy(hbuf.at[pl.ds(0, _PR)], hout.at[pl.ds(p0, _PR)])
        plsc.subcore_barrier()

        seg(hout, acc2)
        plsc.subcore_barrier()
        pltpu.sync_copy(acc2.at[pl.ds(sid * _ZR, _ZR)],
                        a2out.at[pl.ds(sid * _ZR, _ZR)])

    @pl.when(cid == 0)
    def _():
        branch(xrp_hbm, xsp_hbm, ep_hbm, hp_hbm, a2p_hbm)

    @pl.when(cid == 1)
    def _():
        branch(xrl_hbm, xsl_hbm, el_hbm, hl_hbm, a2l_hbm)


def _mlp1_body(xp_ref, xl_ref, wrp, wsp, wrl, wsl, bp, bl,
               xrp_ref, xsp_ref, xrl_ref, xsl_ref):
    xp = xp_ref[...]
    xl = xl_ref[...]
    xrp_ref[...] = jnp.dot(xp, wrp[...], preferred_element_type=_f32)
    xsp_ref[...] = jnp.dot(xp, wsp[...], preferred_element_type=_f32) + bp[...]
    xrl_ref[...] = jnp.dot(xl, wrl[...], preferred_element_type=_f32)
    xsl_ref[...] = jnp.dot(xl, wsl[...], preferred_element_type=_f32) + bl[...]


def _head_body(a2p, hp, a2l, hl, wrp2, wsp2, wrl2, wsl2, b2p, b2l,
               winp, winl, bin_, wout, bout, out_ref):
    ap_t = jnp.dot(wrp2[...], winp[...], preferred_element_type=_f32)
    ap_b = jnp.dot(wsp2[...], winp[...], preferred_element_type=_f32)
    al_t = jnp.dot(wrl2[...], winl[...], preferred_element_type=_f32)
    al_b = jnp.dot(wsl2[...], winl[...], preferred_element_type=_f32)
    c = (jnp.dot(b2p[...], winp[...], preferred_element_type=_f32)
         + jnp.dot(b2l[...], winl[...], preferred_element_type=_f32)
         + bin_[...])
    a = (jnp.dot(a2p[...], ap_t, preferred_element_type=_f32)
         + jnp.dot(hp[...], ap_b, preferred_element_type=_f32)
         + jnp.dot(a2l[...], al_t, preferred_element_type=_f32)
         + jnp.dot(hl[...], al_b, preferred_element_type=_f32)
         + c)
    a = jnp.maximum(a, 0.0)
    out_ref[...] = jnp.tanh(
        jnp.dot(a, wout[...], preferred_element_type=_f32) + bout[...])


def _full(shape):
    return pl.BlockSpec(shape, lambda i: (0, 0))


def _rows(w):
    return pl.BlockSpec((_BLK, w), lambda i: (i, 0))


def kernel(protein_data, protein_edge_index, ligand_data, ligand_edge_index,
           p_Wr1, p_Ws1, p_b1, p_Wr2, p_Ws2, p_b2,
           l_Wr1, l_Ws1, l_b1, l_Wr2, l_Ws2, l_b2,
           W_in, b_in, W_out, b_out):
    ep = protein_edge_index.astype(jnp.int32).reshape(2, _EROWS, _CHUNK)
    el = ligand_edge_index.astype(jnp.int32).reshape(2, _EROWS, _CHUNK)

    nblk = _N // _BLK
    o16 = jax.ShapeDtypeStruct((_N, _HID), _f32)

    xrp, xsp, xrl, xsl = pl.pallas_call(
        _mlp1_body,
        grid=(nblk,),
        in_specs=[_rows(_IN), _rows(_IN),
                  _full((_IN, _HID)), _full((_IN, _HID)),
                  _full((_IN, _HID)), _full((_IN, _HID)),
                  _full((1, _HID)), _full((1, _HID))],
        out_specs=[_rows(_HID)] * 4,
        out_shape=[o16] * 4,
    )(protein_data, ligand_data, p_Wr1, p_Ws1, l_Wr1, l_Ws1,
      p_b1.reshape(1, _HID), l_b1.reshape(1, _HID))

    hp, hl, a2p, a2l = _gnn_sc(xrp, xrl, xsp, xsl, ep, el)

    ogcn = W_in.shape[0] // 2
    ahid = W_in.shape[1]
    act = W_out.shape[1]
    out = pl.pallas_call(
        _head_body,
        grid=(nblk,),
        in_specs=[_rows(_HID)] * 4 + [
            _full((_HID, ogcn)), _full((_HID, ogcn)),
            _full((_HID, ogcn)), _full((_HID, ogcn)),
            _full((1, ogcn)), _full((1, ogcn)),
            _full((ogcn, ahid)), _full((ogcn, ahid)),
            _full((1, ahid)), _full((ahid, act)), _full((1, act))],
        out_specs=_rows(act),
        out_shape=jax.ShapeDtypeStruct((_N, act), _f32),
    )(a2p, hp, a2l, hl, p_Wr2, p_Ws2, l_Wr2, l_Ws2,
      p_b2.reshape(1, ogcn), l_b2.reshape(1, ogcn),
      W_in[:ogcn], W_in[ogcn:], b_in.reshape(1, ahid), W_out,
      b_out.reshape(1, act))
    return out


# speedup vs baseline: 2.0810x; 1.0187x over previous
import functools

import jax
import jax.numpy as jnp
from jax import lax
from jax.experimental import pallas as pl
from jax.experimental.pallas import tpu as pltpu
from jax.experimental.pallas import tpu_sc as plsc

_N = 10000
_IN = 128
_HID = 16
_E = 320000

_CHUNK = 1000
_NMACRO = 20
_EROWS = _E // _CHUNK
_NPAD = 10240
_ZR = _NPAD // 16
_PR = _N // 16

_BLK = 2000

_sc_mesh = plsc.VectorSubcoreMesh(core_axis_name="c", subcore_axis_name="s")

_f32 = jnp.float32


@functools.partial(
    pl.kernel,
    out_type=(jax.ShapeDtypeStruct((_N, _HID), _f32),
              jax.ShapeDtypeStruct((_N, _HID), _f32),
              jax.ShapeDtypeStruct((_NPAD, _HID), _f32),
              jax.ShapeDtypeStruct((_NPAD, _HID), _f32)),
    mesh=_sc_mesh,
    scratch_types=[
        pltpu.VMEM((_NMACRO, _CHUNK), jnp.int32),
        pltpu.VMEM((_NMACRO, _CHUNK), jnp.int32),
        pltpu.VMEM((_CHUNK, _HID), _f32),
        pltpu.VMEM((_CHUNK, _HID), _f32),
        pltpu.VMEM_SHARED((_NPAD, _HID), _f32),
        pltpu.VMEM_SHARED((_NPAD, _HID), _f32),
        pltpu.VMEM_SHARED((_N, _HID), _f32),
        pltpu.SemaphoreType.DMA,
        pltpu.SemaphoreType.DMA,
    ],
    compiler_params=pltpu.CompilerParams(use_tc_tiling_on_sc=False),
)
def _gnn_sc(xrp_hbm, xrl_hbm, xsp_hbm, xsl_hbm, ep_hbm, el_hbm,
            hp_hbm, hl_hbm, a2p_hbm, a2l_hbm,
            sidx, didx, rows_a, rows_b, acc1, acc2, table, gsem, ssem):
    cid = lax.axis_index("c")
    sid = lax.axis_index("s")

    hbuf = rows_a
    xsbuf = rows_b

    zero = jnp.zeros((_HID,), _f32)

    @pl.loop(0, _ZR)
    def _(i):
        hbuf[i, :] = zero

    pltpu.sync_copy(hbuf.at[pl.ds(0, _ZR)], acc1.at[pl.ds(sid * _ZR, _ZR)])
    pltpu.sync_copy(hbuf.at[pl.ds(0, _ZR)], acc2.at[pl.ds(sid * _ZR, _ZR)])
    plsc.subcore_barrier()

    def seg(x_src_ref, acc):
        def g(m, buf):
            return pltpu.async_copy(x_src_ref.at[sidx.at[m]], buf, gsem)

        def s(m, buf):
            return pltpu.async_copy(buf, acc.at[didx.at[m]], ssem, add=True)

        K = 5

        @pl.loop(0, _NMACRO // K)
        def _(mm):
            m0 = K * mm
            bufs = [rows_a, rows_b]
            inflight = {0: g(m0, bufs[0]), 1: g(m0 + 1, bufs[1])}
            for j in range(K):
                buf = bufs[j % 2]
                inflight[j].wait()
                s_h = s(m0 + j, buf)
                s_h.wait()
                if j + 2 < K:
                    inflight[j + 2] = g(m0 + j + 2, buf)

    def branch(xr, xs, edges, hout, a2out):
        base = sid * _NMACRO
        pltpu.sync_copy(edges.at[0].at[pl.ds(base, _NMACRO)], sidx)
        pltpu.sync_copy(edges.at[1].at[pl.ds(base, _NMACRO)], didx)

        t0 = sid * _PR
        pltpu.sync_copy(xr.at[pl.ds(t0, _PR)], rows_b.at[pl.ds(0, _PR)])
        pltpu.sync_copy(rows_b.at[pl.ds(0, _PR)], table.at[pl.ds(t0, _PR)])
        plsc.subcore_barrier()

        seg(table, acc1)
        plsc.subcore_barrier()

        p0 = sid * _PR
        pltpu.sync_copy(acc1.at[pl.ds(p0, _PR)], hbuf.at[pl.ds(0, _PR)])
        pltpu.sync_copy(xs.at[pl.ds(p0, _PR)], xsbuf.at[pl.ds(0, _PR)])

        @pl.loop(0, _PR)
        def _(i):
            hbuf[i, :] = jnp.maximum(hbuf[i, :] + xsbuf[i, :], 0.0)

        pltpu.sync_copy(hbuf.at[pl.ds(0, _PR)], hout.at[pl.ds(p0, _PR)])
        pltpu.sync_copy(hbuf.at[pl.ds(0, _PR)], table.at[pl.ds(p0, _PR)])
        plsc.subcore_barrier()

        seg(table, acc2)
        plsc.subcore_barrier()
        pltpu.sync_copy(acc2.at[pl.ds(sid * _ZR, _ZR)],
                        a2out.at[pl.ds(sid * _ZR, _ZR)])

    @pl.when(cid == 0)
    def _():
        branch(xrp_hbm, xsp_hbm, ep_hbm, hp_hbm, a2p_hbm)

    @pl.when(cid == 1)
    def _():
        branch(xrl_hbm, xsl_hbm, el_hbm, hl_hbm, a2l_hbm)


def _mlp1_body(xp_ref, xl_ref, wrp, wsp, wrl, wsl, bp, bl,
               xrp_ref, xsp_ref, xrl_ref, xsl_ref):
    xp = xp_ref[...]
    xl = xl_ref[...]
    xrp_ref[...] = jnp.dot(xp, wrp[...], preferred_element_type=_f32)
    xsp_ref[...] = jnp.dot(xp, wsp[...], preferred_element_type=_f32) + bp[...]
    xrl_ref[...] = jnp.dot(xl, wrl[...], preferred_element_type=_f32)
    xsl_ref[...] = jnp.dot(xl, wsl[...], preferred_element_type=_f32) + bl[...]


def _head_body(a2p, hp, a2l, hl, wrp2, wsp2, wrl2, wsl2, b2p, b2l,
               winp, winl, bin_, wout, bout, out_ref):
    ap_t = jnp.dot(wrp2[...], winp[...], preferred_element_type=_f32)
    ap_b = jnp.dot(wsp2[...], winp[...], preferred_element_type=_f32)
    al_t = jnp.dot(wrl2[...], winl[...], preferred_element_type=_f32)
    al_b = jnp.dot(wsl2[...], winl[...], preferred_element_type=_f32)
    c = (jnp.dot(b2p[...], winp[...], preferred_element_type=_f32)
         + jnp.dot(b2l[...], winl[...], preferred_element_type=_f32)
         + bin_[...])
    a = (jnp.dot(a2p[...], ap_t, preferred_element_type=_f32)
         + jnp.dot(hp[...], ap_b, preferred_element_type=_f32)
         + jnp.dot(a2l[...], al_t, preferred_element_type=_f32)
         + jnp.dot(hl[...], al_b, preferred_element_type=_f32)
         + c)
    a = jnp.maximum(a, 0.0)
    out_ref[...] = jnp.tanh(
        jnp.dot(a, wout[...], preferred_element_type=_f32) + bout[...])


def _full(shape):
    return pl.BlockSpec(shape, lambda i: (0, 0))


def _rows(w):
    return pl.BlockSpec((_BLK, w), lambda i: (i, 0))


def kernel(protein_data, protein_edge_index, ligand_data, ligand_edge_index,
           p_Wr1, p_Ws1, p_b1, p_Wr2, p_Ws2, p_b2,
           l_Wr1, l_Ws1, l_b1, l_Wr2, l_Ws2, l_b2,
           W_in, b_in, W_out, b_out):
    ep = protein_edge_index.astype(jnp.int32).reshape(2, _EROWS, _CHUNK)
    el = ligand_edge_index.astype(jnp.int32).reshape(2, _EROWS, _CHUNK)

    nblk = _N // _BLK
    o16 = jax.ShapeDtypeStruct((_N, _HID), _f32)

    xrp, xsp, xrl, xsl = pl.pallas_call(
        _mlp1_body,
        grid=(nblk,),
        in_specs=[_rows(_IN), _rows(_IN),
                  _full((_IN, _HID)), _full((_IN, _HID)),
                  _full((_IN, _HID)), _full((_IN, _HID)),
                  _full((1, _HID)), _full((1, _HID))],
        out_specs=[_rows(_HID)] * 4,
        out_shape=[o16] * 4,
    )(protein_data, ligand_data, p_Wr1, p_Ws1, l_Wr1, l_Ws1,
      p_b1.reshape(1, _HID), l_b1.reshape(1, _HID))

    hp, hl, a2p, a2l = _gnn_sc(xrp, xrl, xsp, xsl, ep, el)

    ogcn = W_in.shape[0] // 2
    ahid = W_in.shape[1]
    act = W_out.shape[1]
    out = pl.pallas_call(
        _head_body,
        grid=(nblk,),
        in_specs=[_rows(_HID)] * 4 + [
            _full((_HID, ogcn)), _full((_HID, ogcn)),
            _full((_HID, ogcn)), _full((_HID, ogcn)),
            _full((1, ogcn)), _full((1, ogcn)),
            _full((ogcn, ahid)), _full((ogcn, ahid)),
            _full((1, ahid)), _full((ahid, act)), _full((1, act))],
        out_specs=_rows(act),
        out_shape=jax.ShapeDtypeStruct((_N, act), _f32),
    )(a2p, hp, a2l, hl, p_Wr2, p_Ws2, l_Wr2, l_Ws2,
      p_b2.reshape(1, ogcn), l_b2.reshape(1, ogcn),
      W_in[:ogcn], W_in[ogcn:], b_in.reshape(1, ahid), W_out,
      b_out.reshape(1, act))
    return out


# speedup vs baseline: 2.1279x; 1.0225x over previous
import functools

import jax
import jax.numpy as jnp
from jax import lax
from jax.experimental import pallas as pl
from jax.experimental.pallas import tpu as pltpu
from jax.experimental.pallas import tpu_sc as plsc

_N = 10000
_IN = 128
_HID = 16
_E = 320000

_CHUNK = 1000
_NMACRO = 20
_EROWS = _E // _CHUNK
_NPAD = 10240
_ZR = _NPAD // 16
_PR = _N // 16

_BLK = 2000

_sc_mesh = plsc.VectorSubcoreMesh(core_axis_name="c", subcore_axis_name="s")

_f32 = jnp.float32


@functools.partial(
    pl.kernel,
    out_type=(jax.ShapeDtypeStruct((_N, _HID), _f32),
              jax.ShapeDtypeStruct((_N, _HID), _f32),
              jax.ShapeDtypeStruct((_NPAD, _HID), _f32),
              jax.ShapeDtypeStruct((_NPAD, _HID), _f32)),
    mesh=_sc_mesh,
    scratch_types=[
        pltpu.VMEM((_NMACRO * _CHUNK,), jnp.int32),
        pltpu.VMEM((_NMACRO * _CHUNK,), jnp.int32),
        pltpu.VMEM((_CHUNK, _HID), _f32),
        pltpu.VMEM((_CHUNK, _HID), _f32),
        pltpu.VMEM_SHARED((_NPAD, _HID), _f32),
        pltpu.VMEM_SHARED((_NPAD, _HID), _f32),
        pltpu.VMEM_SHARED((_N, _HID), _f32),
        pltpu.SemaphoreType.DMA,
        pltpu.SemaphoreType.DMA,
    ],
    compiler_params=pltpu.CompilerParams(use_tc_tiling_on_sc=False),
)
def _gnn_sc(xrp_hbm, xrl_hbm, xsp_hbm, xsl_hbm, sp_hbm, dp_hbm, sl_hbm, dl_hbm,
            hp_hbm, hl_hbm, a2p_hbm, a2l_hbm,
            sidx, didx, rows_a, rows_b, acc1, acc2, table, gsem, ssem):
    cid = lax.axis_index("c")
    sid = lax.axis_index("s")

    hbuf = rows_a
    xsbuf = rows_b

    zero = jnp.zeros((_HID,), _f32)

    @pl.loop(0, _ZR)
    def _(i):
        hbuf[i, :] = zero

    pltpu.sync_copy(hbuf.at[pl.ds(0, _ZR)], acc1.at[pl.ds(sid * _ZR, _ZR)])
    pltpu.sync_copy(hbuf.at[pl.ds(0, _ZR)], acc2.at[pl.ds(sid * _ZR, _ZR)])
    plsc.subcore_barrier()

    def seg(x_src_ref, acc):
        def g(m, buf):
            return pltpu.async_copy(
                x_src_ref.at[sidx.at[pl.ds(m * _CHUNK, _CHUNK)]], buf, gsem)

        def s(m, buf):
            return pltpu.async_copy(
                buf, acc.at[didx.at[pl.ds(m * _CHUNK, _CHUNK)]], ssem, add=True)

        K = 5

        @pl.loop(0, _NMACRO // K)
        def _(mm):
            m0 = K * mm
            bufs = [rows_a, rows_b]
            inflight = {0: g(m0, bufs[0]), 1: g(m0 + 1, bufs[1])}
            for j in range(K):
                buf = bufs[j % 2]
                inflight[j].wait()
                s_h = s(m0 + j, buf)
                s_h.wait()
                if j + 2 < K:
                    inflight[j + 2] = g(m0 + j + 2, buf)

    def branch(xr, xs, src, dst, hout, a2out):
        base = sid * _NMACRO * _CHUNK
        pltpu.sync_copy(src.at[pl.ds(base, _NMACRO * _CHUNK)], sidx)
        pltpu.sync_copy(dst.at[pl.ds(base, _NMACRO * _CHUNK)], didx)

        t0 = sid * _PR
        pltpu.sync_copy(xr.at[pl.ds(t0, _PR)], rows_b.at[pl.ds(0, _PR)])
        pltpu.sync_copy(rows_b.at[pl.ds(0, _PR)], table.at[pl.ds(t0, _PR)])
        plsc.subcore_barrier()

        seg(table, acc1)
        plsc.subcore_barrier()

        p0 = sid * _PR
        pltpu.sync_copy(acc1.at[pl.ds(p0, _PR)], hbuf.at[pl.ds(0, _PR)])
        pltpu.sync_copy(xs.at[pl.ds(p0, _PR)], xsbuf.at[pl.ds(0, _PR)])

        @pl.loop(0, _PR)
        def _(i):
            hbuf[i, :] = jnp.maximum(hbuf[i, :] + xsbuf[i, :], 0.0)

        pltpu.sync_copy(hbuf.at[pl.ds(0, _PR)], hout.at[pl.ds(p0, _PR)])
        pltpu.sync_copy(hbuf.at[pl.ds(0, _PR)], table.at[pl.ds(p0, _PR)])
        plsc.subcore_barrier()

        seg(table, acc2)
        plsc.subcore_barrier()
        pltpu.sync_copy(acc2.at[pl.ds(sid * _ZR, _ZR)],
                        a2out.at[pl.ds(sid * _ZR, _ZR)])

    @pl.when(cid == 0)
    def _():
        branch(xrp_hbm, xsp_hbm, sp_hbm, dp_hbm, hp_hbm, a2p_hbm)

    @pl.when(cid == 1)
    def _():
        branch(xrl_hbm, xsl_hbm, sl_hbm, dl_hbm, hl_hbm, a2l_hbm)


def _mlp1_body(xp_ref, xl_ref, wrp, wsp, wrl, wsl, bp, bl,
               xrp_ref, xsp_ref, xrl_ref, xsl_ref):
    xp = xp_ref[...]
    xl = xl_ref[...]
    xrp_ref[...] = jnp.dot(xp, wrp[...], preferred_element_type=_f32)
    xsp_ref[...] = jnp.dot(xp, wsp[...], preferred_element_type=_f32) + bp[...]
    xrl_ref[...] = jnp.dot(xl, wrl[...], preferred_element_type=_f32)
    xsl_ref[...] = jnp.dot(xl, wsl[...], preferred_element_type=_f32) + bl[...]


def _edges_body(ep_ref, el_ref, sp_ref, dp_ref, sl_ref, dl_ref):
    sp_ref[...] = ep_ref[0, :]
    dp_ref[...] = ep_ref[1, :]
    sl_ref[...] = el_ref[0, :]
    dl_ref[...] = el_ref[1, :]


def _head_body(a2p, hp, a2l, hl, wrp2, wsp2, wrl2, wsl2, b2p, b2l,
               winp, winl, bin_, wout, bout, out_ref):
    ap_t = jnp.dot(wrp2[...], winp[...], preferred_element_type=_f32)
    ap_b = jnp.dot(wsp2[...], winp[...], preferred_element_type=_f32)
    al_t = jnp.dot(wrl2[...], winl[...], preferred_element_type=_f32)
    al_b = jnp.dot(wsl2[...], winl[...], preferred_element_type=_f32)
    c = (jnp.dot(b2p[...], winp[...], preferred_element_type=_f32)
         + jnp.dot(b2l[...], winl[...], preferred_element_type=_f32)
         + bin_[...])
    a = (jnp.dot(a2p[...], ap_t, preferred_element_type=_f32)
         + jnp.dot(hp[...], ap_b, preferred_element_type=_f32)
         + jnp.dot(a2l[...], al_t, preferred_element_type=_f32)
         + jnp.dot(hl[...], al_b, preferred_element_type=_f32)
         + c)
    a = jnp.maximum(a, 0.0)
    out_ref[...] = jnp.tanh(
        jnp.dot(a, wout[...], preferred_element_type=_f32) + bout[...])


def _full(shape):
    return pl.BlockSpec(shape, lambda i: (0, 0))


def _rows(w):
    return pl.BlockSpec((_BLK, w), lambda i: (i, 0))


def kernel(protein_data, protein_edge_index, ligand_data, ligand_edge_index,
           p_Wr1, p_Ws1, p_b1, p_Wr2, p_Ws2, p_b2,
           l_Wr1, l_Ws1, l_b1, l_Wr2, l_Ws2, l_b2,
           W_in, b_in, W_out, b_out):
    ep = protein_edge_index.astype(jnp.int32)
    el = ligand_edge_index.astype(jnp.int32)

    nblk = _N // _BLK
    o16 = jax.ShapeDtypeStruct((_N, _HID), _f32)
    oe = jax.ShapeDtypeStruct((_E,), jnp.int32)

    sp, dp, sl, dl = pl.pallas_call(
        _edges_body,
        grid=(1,),
        in_specs=[pl.BlockSpec((2, _E), lambda i: (0, 0))] * 2,
        out_specs=[pl.BlockSpec((_E,), lambda i: (0,))] * 4,
        out_shape=[oe] * 4,
    )(ep, el)

    xrp, xsp, xrl, xsl = pl.pallas_call(
        _mlp1_body,
        grid=(nblk,),
        in_specs=[_rows(_IN), _rows(_IN),
                  _full((_IN, _HID)), _full((_IN, _HID)),
                  _full((_IN, _HID)), _full((_IN, _HID)),
                  _full((1, _HID)), _full((1, _HID))],
        out_specs=[_rows(_HID)] * 4,
        out_shape=[o16] * 4,
    )(protein_data, ligand_data, p_Wr1, p_Ws1, l_Wr1, l_Ws1,
      p_b1.reshape(1, _HID), l_b1.reshape(1, _HID))

    hp, hl, a2p, a2l = _gnn_sc(xrp, xrl, xsp, xsl, sp, dp, sl, dl)

    ogcn = W_in.shape[0] // 2
    ahid = W_in.shape[1]
    act = W_out.shape[1]
    out = pl.pallas_call(
        _head_body,
        grid=(nblk,),
        in_specs=[_rows(_HID)] * 4 + [
            _full((_HID, ogcn)), _full((_HID, ogcn)),
            _full((_HID, ogcn)), _full((_HID, ogcn)),
            _full((1, ogcn)), _full((1, ogcn)),
            _full((ogcn, ahid)), _full((ogcn, ahid)),
            _full((1, ahid)), _full((ahid, act)), _full((1, act))],
        out_specs=_rows(act),
        out_shape=jax.ShapeDtypeStruct((_N, act), _f32),
    )(a2p, hp, a2l, hl, p_Wr2, p_Ws2, l_Wr2, l_Ws2,
      p_b2.reshape(1, ogcn), l_b2.reshape(1, ogcn),
      W_in[:ogcn], W_in[ogcn:], b_in.reshape(1, ahid), W_out,
      b_out.reshape(1, act))
    return out


# speedup vs baseline: 2.2306x; 1.0482x over previous
import functools

import jax
import jax.numpy as jnp
from jax import lax
from jax.experimental import pallas as pl
from jax.experimental.pallas import tpu as pltpu
from jax.experimental.pallas import tpu_sc as plsc

_N = 10000
_IN = 128
_HID = 16
_E = 320000

_CHUNK = 1000
_NMACRO = 20
_EROWS = _E // _CHUNK
_NPAD = 10240
_ZR = _NPAD // 16
_PR = _N // 16

_BLK = 2000

_sc_mesh = plsc.VectorSubcoreMesh(core_axis_name="c", subcore_axis_name="s")

_f32 = jnp.float32


@functools.partial(
    pl.kernel,
    out_type=jax.ShapeDtypeStruct((_NPAD, 4 * _HID), _f32),
    mesh=_sc_mesh,
    scratch_types=[
        pltpu.VMEM((_NMACRO * _CHUNK,), jnp.int32),
        pltpu.VMEM((_NMACRO * _CHUNK,), jnp.int32),
        pltpu.VMEM((_CHUNK, _HID), _f32),
        pltpu.VMEM((_CHUNK, _HID), _f32),
        pltpu.VMEM_SHARED((_NPAD, _HID), _f32),
        pltpu.VMEM_SHARED((_NPAD, _HID), _f32),
        pltpu.VMEM_SHARED((_N, _HID), _f32),
        pltpu.SemaphoreType.DMA,
        pltpu.SemaphoreType.DMA,
    ],
    compiler_params=pltpu.CompilerParams(use_tc_tiling_on_sc=False),
)
def _gnn_sc(xrp_hbm, xrl_hbm, xsp_hbm, xsl_hbm, ep_hbm, el_hbm, out_hbm,
            sidx, didx, rows_a, rows_b, acc1, acc2, table, gsem, ssem):
    cid = lax.axis_index("c")
    sid = lax.axis_index("s")

    hbuf = rows_a
    xsbuf = rows_b

    zero = jnp.zeros((_HID,), _f32)

    @pl.loop(0, _ZR)
    def _(i):
        hbuf[i, :] = zero

    pltpu.sync_copy(hbuf.at[pl.ds(0, _ZR)], acc1.at[pl.ds(sid * _ZR, _ZR)])
    pltpu.sync_copy(hbuf.at[pl.ds(0, _ZR)], acc2.at[pl.ds(sid * _ZR, _ZR)])
    plsc.subcore_barrier()

    def seg(x_src_ref, acc):
        def g(m, buf):
            return pltpu.async_copy(
                x_src_ref.at[sidx.at[pl.ds(m * _CHUNK, _CHUNK)]], buf, gsem)

        def s(m, buf):
            return pltpu.async_copy(
                buf, acc.at[didx.at[pl.ds(m * _CHUNK, _CHUNK)]], ssem, add=True)

        K = 5

        @pl.loop(0, _NMACRO // K)
        def _(mm):
            m0 = K * mm
            bufs = [rows_a, rows_b]
            inflight = {0: g(m0, bufs[0]), 1: g(m0 + 1, bufs[1])}
            for j in range(K):
                buf = bufs[j % 2]
                inflight[j].wait()
                s_h = s(m0 + j, buf)
                s_h.wait()
                if j + 2 < K:
                    inflight[j + 2] = g(m0 + j + 2, buf)

    def branch(xr, xs, edges, col0):
        base = sid * _NMACRO * _CHUNK
        pltpu.sync_copy(edges.at[0].at[pl.ds(base, _NMACRO * _CHUNK)], sidx)
        pltpu.sync_copy(edges.at[1].at[pl.ds(base, _NMACRO * _CHUNK)], didx)

        t0 = sid * _PR
        pltpu.sync_copy(xr.at[pl.ds(t0, _PR)], rows_b.at[pl.ds(0, _PR)])
        pltpu.sync_copy(rows_b.at[pl.ds(0, _PR)], table.at[pl.ds(t0, _PR)])
        plsc.subcore_barrier()

        seg(table, acc1)
        plsc.subcore_barrier()

        p0 = sid * _PR
        pltpu.sync_copy(acc1.at[pl.ds(p0, _PR)], hbuf.at[pl.ds(0, _PR)])
        pltpu.sync_copy(xs.at[pl.ds(p0, _PR)], xsbuf.at[pl.ds(0, _PR)])

        @pl.loop(0, _PR)
        def _(i):
            hbuf[i, :] = jnp.maximum(hbuf[i, :] + xsbuf[i, :], 0.0)

        pltpu.sync_copy(hbuf.at[pl.ds(0, _PR)],
                        out_hbm.at[pl.ds(p0, _PR), pl.ds(col0 + _HID, _HID)])
        pltpu.sync_copy(hbuf.at[pl.ds(0, _PR)], table.at[pl.ds(p0, _PR)])
        plsc.subcore_barrier()

        seg(table, acc2)
        plsc.subcore_barrier()
        pltpu.sync_copy(acc2.at[pl.ds(sid * _ZR, _ZR)],
                        out_hbm.at[pl.ds(sid * _ZR, _ZR), pl.ds(col0, _HID)])

    @pl.when(cid == 0)
    def _():
        branch(xrp_hbm, xsp_hbm, ep_hbm, 0)

    @pl.when(cid == 1)
    def _():
        branch(xrl_hbm, xsl_hbm, el_hbm, 2 * _HID)


def _mlp1_body(xp_ref, xl_ref, wrp, wsp, wrl, wsl, bp, bl,
               xrp_ref, xsp_ref, xrl_ref, xsl_ref):
    xp = xp_ref[...]
    xl = xl_ref[...]
    xrp_ref[...] = jnp.dot(xp, wrp[...], preferred_element_type=_f32)
    xsp_ref[...] = jnp.dot(xp, wsp[...], preferred_element_type=_f32) + bp[...]
    xrl_ref[...] = jnp.dot(xl, wrl[...], preferred_element_type=_f32)
    xsl_ref[...] = jnp.dot(xl, wsl[...], preferred_element_type=_f32) + bl[...]


def _head_body(m64, wrp2, wsp2, wrl2, wsl2, b2p, b2l,
               winp, winl, bin_, wout, bout, out_ref):
    w64 = jnp.concatenate([
        jnp.dot(wrp2[...], winp[...], preferred_element_type=_f32),
        jnp.dot(wsp2[...], winp[...], preferred_element_type=_f32),
        jnp.dot(wrl2[...], winl[...], preferred_element_type=_f32),
        jnp.dot(wsl2[...], winl[...], preferred_element_type=_f32),
    ], axis=0)
    c = (jnp.dot(b2p[...], winp[...], preferred_element_type=_f32)
         + jnp.dot(b2l[...], winl[...], preferred_element_type=_f32)
         + bin_[...])
    a = jnp.dot(m64[...], w64, preferred_element_type=_f32) + c
    a = jnp.maximum(a, 0.0)
    out_ref[...] = jnp.tanh(
        jnp.dot(a, wout[...], preferred_element_type=_f32) + bout[...])


def _full(shape):
    return pl.BlockSpec(shape, lambda i: (0, 0))


def _rows(w):
    return pl.BlockSpec((_BLK, w), lambda i: (i, 0))


def kernel(protein_data, protein_edge_index, ligand_data, ligand_edge_index,
           p_Wr1, p_Ws1, p_b1, p_Wr2, p_Ws2, p_b2,
           l_Wr1, l_Ws1, l_b1, l_Wr2, l_Ws2, l_b2,
           W_in, b_in, W_out, b_out):
    ep = protein_edge_index.astype(jnp.int32)
    el = ligand_edge_index.astype(jnp.int32)

    nblk = _N // _BLK
    o16 = jax.ShapeDtypeStruct((_N, _HID), _f32)

    xrp, xsp, xrl, xsl = pl.pallas_call(
        _mlp1_body,
        grid=(nblk,),
        in_specs=[_rows(_IN), _rows(_IN),
                  _full((_IN, _HID)), _full((_IN, _HID)),
                  _full((_IN, _HID)), _full((_IN, _HID)),
                  _full((1, _HID)), _full((1, _HID))],
        out_specs=[_rows(_HID)] * 4,
        out_shape=[o16] * 4,
    )(protein_data, ligand_data, p_Wr1, p_Ws1, l_Wr1, l_Ws1,
      p_b1.reshape(1, _HID), l_b1.reshape(1, _HID))

    m64 = _gnn_sc(xrp, xrl, xsp, xsl, ep, el)

    ogcn = W_in.shape[0] // 2
    ahid = W_in.shape[1]
    act = W_out.shape[1]
    out = pl.pallas_call(
        _head_body,
        grid=(nblk,),
        in_specs=[_rows(4 * _HID)] + [
            _full((_HID, ogcn)), _full((_HID, ogcn)),
            _full((_HID, ogcn)), _full((_HID, ogcn)),
            _full((1, ogcn)), _full((1, ogcn)),
            _full((ogcn, ahid)), _full((ogcn, ahid)),
            _full((1, ahid)), _full((ahid, act)), _full((1, act))],
        out_specs=_rows(act),
        out_shape=jax.ShapeDtypeStruct((_N, act), _f32),
    )(m64, p_Wr2, p_Ws2, l_Wr2, l_Ws2,
      p_b2.reshape(1, ogcn), l_b2.reshape(1, ogcn),
      W_in[:ogcn], W_in[ogcn:], b_in.reshape(1, ahid), W_out,
      b_out.reshape(1, act))
    return out


# speedup vs baseline: 2.2325x; 1.0008x over previous
import functools

import jax
import jax.numpy as jnp
from jax import lax
from jax.experimental import pallas as pl
from jax.experimental.pallas import tpu as pltpu
from jax.experimental.pallas import tpu_sc as plsc

_N = 10000
_IN = 128
_HID = 16
_E = 320000

_CHUNK = 1000
_NMACRO = 20
_NPAD = 10240
_ZR = _NPAD // 16
_PR = _N // 16

_BLK = 2000

_sc_mesh = plsc.VectorSubcoreMesh(core_axis_name="c", subcore_axis_name="s")

_f32 = jnp.float32


@functools.partial(
    pl.kernel,
    out_type=jax.ShapeDtypeStruct((_NPAD, 4 * _HID), _f32),
    mesh=_sc_mesh,
    scratch_types=[
        pltpu.VMEM((_NMACRO * _CHUNK,), jnp.int32),
        pltpu.VMEM((_NMACRO * _CHUNK,), jnp.int32),
        pltpu.VMEM((_CHUNK, _HID), _f32),
        pltpu.VMEM((_CHUNK, _HID), _f32),
        pltpu.VMEM_SHARED((_NPAD, _HID), _f32),
        pltpu.VMEM_SHARED((_NPAD, _HID), _f32),
        pltpu.VMEM_SHARED((_N, _HID), _f32),
        pltpu.SemaphoreType.DMA,
        pltpu.SemaphoreType.DMA,
    ],
    compiler_params=pltpu.CompilerParams(use_tc_tiling_on_sc=False),
)
def _gnn_sc(xrp_hbm, xrl_hbm, xsp_hbm, xsl_hbm, ep_hbm, el_hbm, out_hbm,
            sidx, didx, rows_a, rows_b, acc1, acc2, table, gsem, ssem):
    cid = lax.axis_index("c")
    sid = lax.axis_index("s")

    hbuf = rows_a
    xsbuf = rows_b

    zero = jnp.zeros((_HID,), _f32)

    @pl.loop(0, _ZR)
    def _(i):
        hbuf[i, :] = zero

    pltpu.sync_copy(hbuf.at[pl.ds(0, _ZR)], acc1.at[pl.ds(sid * _ZR, _ZR)])
    pltpu.sync_copy(hbuf.at[pl.ds(0, _ZR)], acc2.at[pl.ds(sid * _ZR, _ZR)])
    plsc.subcore_barrier()

    def seg(x_src_ref, acc):
        def g(m, buf):
            return pltpu.async_copy(
                x_src_ref.at[sidx.at[pl.ds(m * _CHUNK, _CHUNK)]], buf, gsem)

        def s(m, buf):
            return pltpu.async_copy(
                buf, acc.at[didx.at[pl.ds(m * _CHUNK, _CHUNK)]], ssem, add=True)

        K = 5

        @pl.loop(0, _NMACRO // K)
        def _(mm):
            m0 = K * mm
            bufs = [rows_a, rows_b]
            inflight = {0: g(m0, bufs[0]), 1: g(m0 + 1, bufs[1])}
            for j in range(K):
                buf = bufs[j % 2]
                inflight[j].wait()
                s_h = s(m0 + j, buf)
                s_h.wait()
                if j + 2 < K:
                    inflight[j + 2] = g(m0 + j + 2, buf)

    def branch(xr, xs, edges, col0):
        base = sid * _NMACRO * _CHUNK
        pltpu.sync_copy(edges.at[0].at[pl.ds(base, _NMACRO * _CHUNK)], sidx)
        pltpu.sync_copy(edges.at[1].at[pl.ds(base, _NMACRO * _CHUNK)], didx)

        t0 = sid * _PR
        pltpu.sync_copy(xr.at[pl.ds(t0, _PR)], rows_b.at[pl.ds(0, _PR)])
        pltpu.sync_copy(rows_b.at[pl.ds(0, _PR)], table.at[pl.ds(t0, _PR)])
        plsc.subcore_barrier()

        seg(table, acc1)
        plsc.subcore_barrier()

        p0 = sid * _PR
        pltpu.sync_copy(acc1.at[pl.ds(p0, _PR)], hbuf.at[pl.ds(0, _PR)])
        pltpu.sync_copy(xs.at[pl.ds(p0, _PR)], xsbuf.at[pl.ds(0, _PR)])

        @pl.loop(0, _PR)
        def _(i):
            hbuf[i, :] = jnp.maximum(hbuf[i, :] + xsbuf[i, :], 0.0)

        pltpu.sync_copy(hbuf.at[pl.ds(0, _PR)],
                        out_hbm.at[pl.ds(p0, _PR), pl.ds(col0 + _HID, _HID)])
        pltpu.sync_copy(hbuf.at[pl.ds(0, _PR)], table.at[pl.ds(p0, _PR)])
        plsc.subcore_barrier()

        seg(table, acc2)
        plsc.subcore_barrier()
        pltpu.sync_copy(acc2.at[pl.ds(sid * _ZR, _ZR)],
                        out_hbm.at[pl.ds(sid * _ZR, _ZR), pl.ds(col0, _HID)])

    @pl.when(cid == 0)
    def _():
        branch(xrp_hbm, xsp_hbm, ep_hbm, 0)

    @pl.when(cid == 1)
    def _():
        branch(xrl_hbm, xsl_hbm, el_hbm, 2 * _HID)


def _mlp1_body(xp_ref, xl_ref, wrp, wsp, wrl, wsl, bp, bl,
               xrp_ref, xsp_ref, xrl_ref, xsl_ref):
    xp = xp_ref[...]
    xl = xl_ref[...]
    xrp_ref[...] = jnp.dot(xp, wrp[...], preferred_element_type=_f32)
    xsp_ref[...] = jnp.dot(xp, wsp[...], preferred_element_type=_f32) + bp[...]
    xrl_ref[...] = jnp.dot(xl, wrl[...], preferred_element_type=_f32)
    xsl_ref[...] = jnp.dot(xl, wsl[...], preferred_element_type=_f32) + bl[...]


def _head_body(m64, wrp2, wsp2, wrl2, wsl2, b2p, b2l,
               winp, winl, bin_, wout, bout, out_ref):
    w64 = jnp.concatenate([
        jnp.dot(wrp2[...], winp[...], preferred_element_type=_f32),
        jnp.dot(wsp2[...], winp[...], preferred_element_type=_f32),
        jnp.dot(wrl2[...], winl[...], preferred_element_type=_f32),
        jnp.dot(wsl2[...], winl[...], preferred_element_type=_f32),
    ], axis=0)
    c = (jnp.dot(b2p[...], winp[...], preferred_element_type=_f32)
         + jnp.dot(b2l[...], winl[...], preferred_element_type=_f32)
         + bin_[...])
    a = jnp.dot(m64[...], w64, preferred_element_type=_f32) + c
    a = jnp.maximum(a, 0.0)
    out_ref[...] = jnp.tanh(
        jnp.dot(a, wout[...], preferred_element_type=_f32) + bout[...])


def _full(shape):
    return pl.BlockSpec(shape, lambda i: (0, 0))


def _rows(w):
    return pl.BlockSpec((_BLK, w), lambda i: (i, 0))


def kernel(protein_data, protein_edge_index, ligand_data, ligand_edge_index,
           p_Wr1, p_Ws1, p_b1, p_Wr2, p_Ws2, p_b2,
           l_Wr1, l_Ws1, l_b1, l_Wr2, l_Ws2, l_b2,
           W_in, b_in, W_out, b_out):
    ep = protein_edge_index.astype(jnp.int32)
    el = ligand_edge_index.astype(jnp.int32)

    nblk = _N // _BLK
    o16 = jax.ShapeDtypeStruct((_N, _HID), _f32)

    xrp, xsp, xrl, xsl = pl.pallas_call(
        _mlp1_body,
        grid=(nblk,),
        in_specs=[_rows(_IN), _rows(_IN),
                  _full((_IN, _HID)), _full((_IN, _HID)),
                  _full((_IN, _HID)), _full((_IN, _HID)),
                  _full((1, _HID)), _full((1, _HID))],
        out_specs=[_rows(_HID)] * 4,
        out_shape=[o16] * 4,
    )(protein_data, ligand_data, p_Wr1, p_Ws1, l_Wr1, l_Ws1,
      p_b1.reshape(1, _HID), l_b1.reshape(1, _HID))

    m64 = _gnn_sc(xrp, xrl, xsp, xsl, ep, el)

    ogcn = W_in.shape[0] // 2
    ahid = W_in.shape[1]
    act = W_out.shape[1]
    out = pl.pallas_call(
        _head_body,
        grid=(nblk,),
        in_specs=[_rows(4 * _HID)] + [
            _full((_HID, ogcn)), _full((_HID, ogcn)),
            _full((_HID, ogcn)), _full((_HID, ogcn)),
            _full((1, ogcn)), _full((1, ogcn)),
            _full((ogcn, ahid)), _full((ogcn, ahid)),
            _full((1, ahid)), _full((ahid, act)), _full((1, act))],
        out_specs=_rows(act),
        out_shape=jax.ShapeDtypeStruct((_N, act), _f32),
    )(m64, p_Wr2, p_Ws2, l_Wr2, l_Ws2,
      p_b2.reshape(1, ogcn), l_b2.reshape(1, ogcn),
      W_in[:ogcn], W_in[ogcn:], b_in.reshape(1, ahid), W_out,
      b_out.reshape(1, act))
    return out
